# Initial kernel scaffold; baseline (speedup 1.0000x reference)
import jax
import jax.numpy as jnp
from jax.experimental import pallas as pl


def kernel(x, edge_index, batch, W1, b1, W2, b2, Wl, bl):
    raise NotImplementedError("write your pallas kernel here")


# speedup vs baseline: 27.7536x; 27.7536x over previous
import functools

import jax
import jax.numpy as jnp
from jax import lax
from jax.experimental import pallas as pl
from jax.experimental.pallas import tpu as pltpu
from jax.experimental.pallas import tpu_sc as plsc

N = 10000
E = 320000
F_IN = 128
H = 64
G = 64

NC = 2
NS = 16
NW = NC * NS
BLK = 128
EPW = E // NW
NBLK = -(-EPW // BLK)
EPAD = NW * NBLK * BLK
N_PAD = 10240
RPT = N_PAD // NS
DH = 16
RB = 1280

@functools.lru_cache(maxsize=1)
def _sc_kernels():
    mesh = plsc.VectorSubcoreMesh(
        core_axis_name="c", subcore_axis_name="s", num_cores=NC, num_subcores=NS
    )

    @functools.partial(
        pl.kernel,
        out_type=jax.ShapeDtypeStruct((NC, N_PAD, DH), jnp.float32),
        mesh=mesh,
        scratch_types=[
            pltpu.VMEM_SHARED((N_PAD, DH), jnp.float32),
            pltpu.VMEM((NBLK, BLK), jnp.int32),
            pltpu.VMEM((BLK, DH), jnp.float32),
            pltpu.VMEM((16, DH), jnp.float32),
            pltpu.VMEM((RPT, DH), jnp.float32),
        ],
        compiler_params=pltpu.CompilerParams(use_tc_tiling_on_sc=False),
    )
    def sc_hist(dstb_hbm, ones_hbm, zeros_hbm, out_hbm, acc_sh,
                dst_v, ones_v, zb_v, rd_v):
        c = lax.axis_index("c")
        s = lax.axis_index("s")
        wid = c * NS + s

        pltpu.sync_copy(ones_hbm, ones_v)
        pltpu.sync_copy(zeros_hbm, zb_v)
        pltpu.sync_copy(dstb_hbm.at[wid], dst_v)

        @pl.loop(0, RPT // 16)
        def _zero(k):
            pltpu.sync_copy(zb_v, acc_sh.at[pl.ds(s * RPT + k * 16, 16)])

        plsc.subcore_barrier()

        @pl.loop(0, NBLK)
        def _scatter(j):
            pltpu.sync_copy(ones_v, acc_sh.at[dst_v.at[j]], add=True)

        plsc.subcore_barrier()

        pltpu.sync_copy(acc_sh.at[pl.ds(s * RPT, RPT)], rd_v)
        pltpu.sync_copy(rd_v, out_hbm.at[c].at[pl.ds(s * RPT, RPT)])

    @functools.partial(
        pl.kernel,
        out_type=jax.ShapeDtypeStruct((NC, N_PAD, H), jnp.float32),
        mesh=mesh,
        scratch_types=[
            pltpu.VMEM_SHARED((N_PAD, H), jnp.float32),
            pltpu.VMEM((NBLK, BLK), jnp.int32),
            pltpu.VMEM((NBLK, BLK), jnp.int32),
            pltpu.VMEM((BLK, H), jnp.float32),
            pltpu.VMEM((16, H), jnp.float32),
            pltpu.VMEM((RPT, H), jnp.float32),
            pltpu.SemaphoreType.DMA,
        ],
        compiler_params=pltpu.CompilerParams(use_tc_tiling_on_sc=False),
    )
    def sc_agg(g_hbm, srcb_hbm, dstb_hbm, zeros_hbm, out_hbm, acc_sh,
               src_v, dst_v, rows_v, zb_v, rd_v, sem):
        c = lax.axis_index("c")
        s = lax.axis_index("s")
        wid = c * NS + s

        pltpu.sync_copy(zeros_hbm, zb_v)
        pltpu.sync_copy(srcb_hbm.at[wid], src_v)
        pltpu.sync_copy(dstb_hbm.at[wid], dst_v)

        @pl.loop(0, RPT // 16)
        def _zero(k):
            pltpu.sync_copy(zb_v, acc_sh.at[pl.ds(s * RPT + k * 16, 16)])

        plsc.subcore_barrier()

        @pl.loop(0, NBLK)
        def _edges(j):
            pltpu.async_copy(g_hbm.at[src_v.at[j]], rows_v, sem).wait()
            pltpu.sync_copy(rows_v, acc_sh.at[dst_v.at[j]], add=True)

        plsc.subcore_barrier()

        pltpu.sync_copy(acc_sh.at[pl.ds(s * RPT, RPT)], rd_v)
        pltpu.sync_copy(rd_v, out_hbm.at[c].at[pl.ds(s * RPT, RPT)])

    return sc_hist, sc_agg


def _dinv_from_degp(degp_blk):
    deg = degp_blk[0, :, 0:1] + degp_blk[1, :, 0:1] + 1.0
    return lax.rsqrt(deg)


def _tc_prep1_body(x_ref, w1_ref, degp_ref, g1_ref):
    h = jnp.dot(x_ref[...], w1_ref[...], preferred_element_type=jnp.float32)
    g1_ref[...] = h * _dinv_from_degp(degp_ref[...])


def _tc_mid_body(p_ref, g1_ref, degp_ref, b1_ref, w2_ref, g2_ref):
    dinv = _dinv_from_degp(degp_ref[...])
    acc = p_ref[0] + p_ref[1] + g1_ref[...]
    out1 = jnp.maximum(acc * dinv + b1_ref[...], 0.0)
    h2 = jnp.dot(out1, w2_ref[...], preferred_element_type=jnp.float32)
    g2_ref[...] = h2 * dinv


def _tc_final_body(p_ref, g2_ref, degp_ref, b2_ref, batch_ref, wl_ref, bl_ref,
                   out_ref, sums_ref, cnt_ref):
    i = pl.program_id(0)
    dinv = _dinv_from_degp(degp_ref[...])
    acc = p_ref[0] + p_ref[1] + g2_ref[...]
    out2 = acc * dinv + b2_ref[...]

    rows = i * RB + lax.broadcasted_iota(jnp.int32, (RB, 1), 0)
    valid = rows < N
    out2m = jnp.where(valid, out2, 0.0)
    vones = jnp.where(valid, 1.0, 0.0)

    gids = lax.broadcasted_iota(jnp.int32, (RB, G), 1)
    oh = jnp.where((batch_ref[...] == gids) & valid, 1.0, 0.0)

    dn = (((0,), (0,)), ((), ()))
    part = lax.dot_general(oh, out2m, dn, preferred_element_type=jnp.float32)
    cntp = lax.dot_general(oh, vones, dn, preferred_element_type=jnp.float32)

    @pl.when(i == 0)
    def _init():
        sums_ref[...] = jnp.zeros_like(sums_ref)
        cnt_ref[...] = jnp.zeros_like(cnt_ref)

    sums_ref[...] += part
    cnt_ref[:, 0:1] += cntp

    @pl.when(i == (N_PAD // RB) - 1)
    def _fin():
        pooled = sums_ref[...] / jnp.maximum(cnt_ref[:, 0:1], 1.0)
        res = jnp.dot(pooled, wl_ref[...], preferred_element_type=jnp.float32)
        out_ref[...] = res + bl_ref[...]


def _row_spec(shape_tail):
    return pl.BlockSpec((RB,) + shape_tail, lambda i: (i,) + (0,) * len(shape_tail))


def _full_spec(shape):
    nd = len(shape)
    return pl.BlockSpec(shape, lambda i: (0,) * nd)


_degp_spec = pl.BlockSpec((2, RB, DH), lambda i: (0, i, 0))
_p_spec = pl.BlockSpec((2, RB, H), lambda i: (0, i, 0))
_grid = (N_PAD // RB,)


def kernel(x, edge_index, batch, W1, b1, W2, b2, Wl, bl):
    src = edge_index[0].astype(jnp.int32)
    dst = edge_index[1].astype(jnp.int32)
    batch2d = batch.astype(jnp.int32)[:, None]

    npad = EPAD - E
    pad_ids = jnp.arange(npad, dtype=jnp.int32)
    src_p = jnp.concatenate([src, pad_ids % N]).reshape(NW, NBLK, BLK)
    dst_p = jnp.concatenate([dst, N + pad_ids % (N_PAD - N)]).reshape(NW, NBLK, BLK)

    ones_upd = jnp.ones((BLK, DH), jnp.float32)
    zeros_h = jnp.zeros((16, DH), jnp.float32)
    zeros_a = jnp.zeros((16, H), jnp.float32)

    sc_hist, sc_agg = _sc_kernels()
    degp = sc_hist(dst_p, ones_upd, zeros_h)

    g1 = pl.pallas_call(
        _tc_prep1_body,
        grid=_grid,
        in_specs=[
            _row_spec((F_IN,)),
            _full_spec((F_IN, H)),
            _degp_spec,
        ],
        out_specs=_row_spec((H,)),
        out_shape=jax.ShapeDtypeStruct((N_PAD, H), jnp.float32),
    )(x, W1, degp)

    p1 = sc_agg(g1, src_p, dst_p, zeros_a)

    g2 = pl.pallas_call(
        _tc_mid_body,
        grid=_grid,
        in_specs=[
            _p_spec,
            _row_spec((H,)),
            _degp_spec,
            _full_spec((1, H)),
            _full_spec((H, H)),
        ],
        out_specs=_row_spec((H,)),
        out_shape=jax.ShapeDtypeStruct((N_PAD, H), jnp.float32),
    )(p1, g1, degp, b1[None, :], W2)

    p2 = sc_agg(g2, src_p, dst_p, zeros_a)

    out = pl.pallas_call(
        _tc_final_body,
        grid=_grid,
        in_specs=[
            _p_spec,
            _row_spec((H,)),
            _degp_spec,
            _full_spec((1, H)),
            _row_spec((1,)),
            _full_spec((H, 1)),
            _full_spec((1, 1)),
        ],
        out_specs=_full_spec((G, 1)),
        out_shape=jax.ShapeDtypeStruct((G, 1), jnp.float32),
        scratch_shapes=[
            pltpu.VMEM((G, G), jnp.float32),
            pltpu.VMEM((G, 8), jnp.float32),
        ],
    )(p2, g2, degp, b2[None, :], batch2d, Wl, bl[:, None])

    return out


# speedup vs baseline: 33.2547x; 1.1982x over previous
import functools

import jax
import jax.numpy as jnp
from jax import lax
from jax.experimental import pallas as pl
from jax.experimental.pallas import tpu as pltpu
from jax.experimental.pallas import tpu_sc as plsc

N = 10000
E = 320000
F_IN = 128
H = 64
G = 64

NC = 2
NS = 16
NW = NC * NS
BLK = 128
EPW = E // NW
NBUF = 2
NBLK = 80
EPAD = NW * NBLK * BLK
N_PAD = 10240
RPT = N_PAD // NS
DH = 16
RB = 1280

@functools.lru_cache(maxsize=1)
def _sc_kernels():
    mesh = plsc.VectorSubcoreMesh(
        core_axis_name="c", subcore_axis_name="s", num_cores=NC, num_subcores=NS
    )

    @functools.partial(
        pl.kernel,
        out_type=jax.ShapeDtypeStruct((NC, N_PAD, DH), jnp.float32),
        mesh=mesh,
        scratch_types=[
            pltpu.VMEM_SHARED((N_PAD, DH), jnp.float32),
            pltpu.VMEM((NBLK, BLK), jnp.int32),
            pltpu.VMEM((BLK, DH), jnp.float32),
            pltpu.VMEM((16, DH), jnp.float32),
            pltpu.VMEM((RPT, DH), jnp.float32),
        ],
        compiler_params=pltpu.CompilerParams(use_tc_tiling_on_sc=False),
    )
    def sc_hist(dstb_hbm, ones_hbm, zeros_hbm, out_hbm, acc_sh,
                dst_v, ones_v, zb_v, rd_v):
        c = lax.axis_index("c")
        s = lax.axis_index("s")
        wid = c * NS + s

        pltpu.sync_copy(ones_hbm, ones_v)
        pltpu.sync_copy(zeros_hbm, zb_v)
        pltpu.sync_copy(dstb_hbm.at[wid], dst_v)

        @pl.loop(0, RPT // 16)
        def _zero(k):
            pltpu.sync_copy(zb_v, acc_sh.at[pl.ds(s * RPT + k * 16, 16)])

        plsc.subcore_barrier()

        @pl.loop(0, NBLK)
        def _scatter(j):
            pltpu.sync_copy(ones_v, acc_sh.at[dst_v.at[j]], add=True)

        plsc.subcore_barrier()

        pltpu.sync_copy(acc_sh.at[pl.ds(s * RPT, RPT)], rd_v)
        pltpu.sync_copy(rd_v, out_hbm.at[c].at[pl.ds(s * RPT, RPT)])

    @functools.partial(
        pl.kernel,
        out_type=jax.ShapeDtypeStruct((NC, N_PAD, H), jnp.float32),
        mesh=mesh,
        scratch_types=[
            pltpu.VMEM_SHARED((N_PAD, H), jnp.float32),
            pltpu.VMEM((NBLK, BLK), jnp.int32),
            pltpu.VMEM((NBLK, BLK), jnp.int32),
            pltpu.VMEM((16, H), jnp.float32),
            pltpu.VMEM((RPT, H), jnp.float32),
        ]
        + [pltpu.VMEM((BLK, H), jnp.float32) for _ in range(NBUF)]
        + [pltpu.SemaphoreType.DMA for _ in range(2 * NBUF)],
        compiler_params=pltpu.CompilerParams(use_tc_tiling_on_sc=False),
    )
    def sc_agg(g_hbm, srcb_hbm, dstb_hbm, zeros_hbm, out_hbm, acc_sh,
               src_v, dst_v, zb_v, rd_v, *bufs_sems):
        bufs = bufs_sems[:NBUF]
        gsem = bufs_sems[NBUF:2 * NBUF]
        ssem = bufs_sems[2 * NBUF:]
        c = lax.axis_index("c")
        s = lax.axis_index("s")
        wid = c * NS + s

        pltpu.sync_copy(zeros_hbm, zb_v)
        pltpu.sync_copy(srcb_hbm.at[wid], src_v)
        pltpu.sync_copy(dstb_hbm.at[wid], dst_v)

        @pl.loop(0, RPT // 16)
        def _zero(k):
            pltpu.sync_copy(zb_v, acc_sh.at[pl.ds(s * RPT + k * 16, 16)])

        plsc.subcore_barrier()

        def gather_start(j, b):
            pltpu.async_copy(g_hbm.at[src_v.at[j]], bufs[b], gsem[b])

        def gather_wait(b):
            pltpu.make_async_copy(g_hbm.at[src_v.at[0]], bufs[b], gsem[b]).wait()

        def scat_start(j, b):
            pltpu.async_copy(bufs[b], acc_sh.at[dst_v.at[j]], ssem[b], add=True)

        def scat_wait(b):
            pltpu.make_async_copy(
                bufs[b], acc_sh.at[dst_v.at[0]], ssem[b]
            ).wait()

        for b in range(NBUF):
            gather_start(b, b)

        @pl.loop(0, NBLK // NBUF - 1)
        def _edges(jg):
            j0 = jg * NBUF
            for b in range(NBUF):
                gather_wait(b)
                scat_start(j0 + b, b)
            for b in range(NBUF):
                scat_wait(b)
                gather_start(j0 + NBUF + b, b)

        for b in range(NBUF):
            gather_wait(b)
            scat_start(NBLK - NBUF + b, b)
        for b in range(NBUF):
            scat_wait(b)

        plsc.subcore_barrier()

        pltpu.sync_copy(acc_sh.at[pl.ds(s * RPT, RPT)], rd_v)
        pltpu.sync_copy(rd_v, out_hbm.at[c].at[pl.ds(s * RPT, RPT)])

    return sc_hist, sc_agg


def _dinv_from_degp(degp_blk):
    deg = degp_blk[0, :, 0:1] + degp_blk[1, :, 0:1] + 1.0
    return lax.rsqrt(deg)


def _tc_prep1_body(x_ref, w1_ref, degp_ref, g1_ref):
    h = jnp.dot(x_ref[...], w1_ref[...], preferred_element_type=jnp.float32)
    g1_ref[...] = h * _dinv_from_degp(degp_ref[...])


def _tc_mid_body(p_ref, g1_ref, degp_ref, b1_ref, w2_ref, g2_ref):
    dinv = _dinv_from_degp(degp_ref[...])
    acc = p_ref[0] + p_ref[1] + g1_ref[...]
    out1 = jnp.maximum(acc * dinv + b1_ref[...], 0.0)
    h2 = jnp.dot(out1, w2_ref[...], preferred_element_type=jnp.float32)
    g2_ref[...] = h2 * dinv


def _tc_final_body(p_ref, g2_ref, degp_ref, b2_ref, batch_ref, wl_ref, bl_ref,
                   out_ref, sums_ref, cnt_ref):
    i = pl.program_id(0)
    dinv = _dinv_from_degp(degp_ref[...])
    acc = p_ref[0] + p_ref[1] + g2_ref[...]
    out2 = acc * dinv + b2_ref[...]

    rows = i * RB + lax.broadcasted_iota(jnp.int32, (RB, 1), 0)
    valid = rows < N
    out2m = jnp.where(valid, out2, 0.0)
    vones = jnp.where(valid, 1.0, 0.0)

    gids = lax.broadcasted_iota(jnp.int32, (RB, G), 1)
    oh = jnp.where((batch_ref[...] == gids) & valid, 1.0, 0.0)

    dn = (((0,), (0,)), ((), ()))
    part = lax.dot_general(oh, out2m, dn, preferred_element_type=jnp.float32)
    cntp = lax.dot_general(oh, vones, dn, preferred_element_type=jnp.float32)

    @pl.when(i == 0)
    def _init():
        sums_ref[...] = jnp.zeros_like(sums_ref)
        cnt_ref[...] = jnp.zeros_like(cnt_ref)

    sums_ref[...] += part
    cnt_ref[:, 0:1] += cntp

    @pl.when(i == (N_PAD // RB) - 1)
    def _fin():
        pooled = sums_ref[...] / jnp.maximum(cnt_ref[:, 0:1], 1.0)
        res = jnp.dot(pooled, wl_ref[...], preferred_element_type=jnp.float32)
        out_ref[...] = res + bl_ref[...]


def _row_spec(shape_tail):
    return pl.BlockSpec((RB,) + shape_tail, lambda i: (i,) + (0,) * len(shape_tail))


def _full_spec(shape):
    nd = len(shape)
    return pl.BlockSpec(shape, lambda i: (0,) * nd)


_degp_spec = pl.BlockSpec((2, RB, DH), lambda i: (0, i, 0))
_p_spec = pl.BlockSpec((2, RB, H), lambda i: (0, i, 0))
_grid = (N_PAD // RB,)


def kernel(x, edge_index, batch, W1, b1, W2, b2, Wl, bl):
    src = edge_index[0].astype(jnp.int32)
    dst = edge_index[1].astype(jnp.int32)
    batch2d = batch.astype(jnp.int32)[:, None]

    npad = EPAD - E
    pad_ids = jnp.arange(npad, dtype=jnp.int32)
    src_p = jnp.concatenate([src, pad_ids % N]).reshape(NW, NBLK, BLK)
    dst_p = jnp.concatenate([dst, N + pad_ids % (N_PAD - N)]).reshape(NW, NBLK, BLK)

    ones_upd = jnp.ones((BLK, DH), jnp.float32)
    zeros_h = jnp.zeros((16, DH), jnp.float32)
    zeros_a = jnp.zeros((16, H), jnp.float32)

    sc_hist, sc_agg = _sc_kernels()
    degp = sc_hist(dst_p, ones_upd, zeros_h)

    g1 = pl.pallas_call(
        _tc_prep1_body,
        grid=_grid,
        in_specs=[
            _row_spec((F_IN,)),
            _full_spec((F_IN, H)),
            _degp_spec,
        ],
        out_specs=_row_spec((H,)),
        out_shape=jax.ShapeDtypeStruct((N_PAD, H), jnp.float32),
    )(x, W1, degp)

    p1 = sc_agg(g1, src_p, dst_p, zeros_a)

    g2 = pl.pallas_call(
        _tc_mid_body,
        grid=_grid,
        in_specs=[
            _p_spec,
            _row_spec((H,)),
            _degp_spec,
            _full_spec((1, H)),
            _full_spec((H, H)),
        ],
        out_specs=_row_spec((H,)),
        out_shape=jax.ShapeDtypeStruct((N_PAD, H), jnp.float32),
    )(p1, g1, degp, b1[None, :], W2)

    p2 = sc_agg(g2, src_p, dst_p, zeros_a)

    out = pl.pallas_call(
        _tc_final_body,
        grid=_grid,
        in_specs=[
            _p_spec,
            _row_spec((H,)),
            _degp_spec,
            _full_spec((1, H)),
            _row_spec((1,)),
            _full_spec((H, 1)),
            _full_spec((1, 1)),
        ],
        out_specs=_full_spec((G, 1)),
        out_shape=jax.ShapeDtypeStruct((G, 1), jnp.float32),
        scratch_shapes=[
            pltpu.VMEM((G, G), jnp.float32),
            pltpu.VMEM((G, 8), jnp.float32),
        ],
    )(p2, g2, degp, b2[None, :], batch2d, Wl, bl[:, None])

    return out


# speedup vs baseline: 40.0749x; 1.2051x over previous
import functools

import jax
import jax.numpy as jnp
from jax import lax
from jax.experimental import pallas as pl
from jax.experimental.pallas import tpu as pltpu
from jax.experimental.pallas import tpu_sc as plsc

N = 10000
E = 320000
F_IN = 128
H = 64
G = 64

NC = 2
NS = 16
NW = NC * NS
BLK = 128
EPW = E // NW
NBUF = 4
NBLK = 80
EPAD = NW * NBLK * BLK
N_PAD = 10240
RPT = N_PAD // NS
DH = 16
RB = 1280

@functools.lru_cache(maxsize=1)
def _sc_kernels():
    mesh = plsc.VectorSubcoreMesh(
        core_axis_name="c", subcore_axis_name="s", num_cores=NC, num_subcores=NS
    )

    @functools.partial(
        pl.kernel,
        out_type=jax.ShapeDtypeStruct((NC, N_PAD, DH), jnp.float32),
        mesh=mesh,
        scratch_types=[
            pltpu.VMEM_SHARED((N_PAD, DH), jnp.float32),
            pltpu.VMEM((NBLK, BLK), jnp.int32),
            pltpu.VMEM((BLK, DH), jnp.float32),
            pltpu.VMEM((16, DH), jnp.float32),
            pltpu.VMEM((RPT, DH), jnp.float32),
        ],
        compiler_params=pltpu.CompilerParams(use_tc_tiling_on_sc=False),
    )
    def sc_hist(dstb_hbm, ones_hbm, zeros_hbm, out_hbm, acc_sh,
                dst_v, ones_v, zb_v, rd_v):
        c = lax.axis_index("c")
        s = lax.axis_index("s")
        wid = c * NS + s

        pltpu.sync_copy(ones_hbm, ones_v)
        pltpu.sync_copy(zeros_hbm, zb_v)
        pltpu.sync_copy(dstb_hbm.at[wid], dst_v)

        @pl.loop(0, RPT // 16)
        def _zero(k):
            pltpu.sync_copy(zb_v, acc_sh.at[pl.ds(s * RPT + k * 16, 16)])

        plsc.subcore_barrier()

        @pl.loop(0, NBLK)
        def _scatter(j):
            pltpu.sync_copy(ones_v, acc_sh.at[dst_v.at[j]], add=True)

        plsc.subcore_barrier()

        pltpu.sync_copy(acc_sh.at[pl.ds(s * RPT, RPT)], rd_v)
        pltpu.sync_copy(rd_v, out_hbm.at[c].at[pl.ds(s * RPT, RPT)])

    @functools.partial(
        pl.kernel,
        out_type=jax.ShapeDtypeStruct((NC, N_PAD, H), jnp.float32),
        mesh=mesh,
        scratch_types=[
            pltpu.VMEM_SHARED((N_PAD, H), jnp.float32),
            pltpu.VMEM((NBLK, BLK), jnp.int32),
            pltpu.VMEM((NBLK, BLK), jnp.int32),
            pltpu.VMEM((16, H), jnp.float32),
            pltpu.VMEM((RPT // 4, H), jnp.float32),
        ]
        + [pltpu.VMEM((BLK, H), jnp.float32) for _ in range(NBUF)]
        + [pltpu.SemaphoreType.DMA for _ in range(2 * NBUF)],
        compiler_params=pltpu.CompilerParams(use_tc_tiling_on_sc=False),
    )
    def sc_agg(g_hbm, srcb_hbm, dstb_hbm, zeros_hbm, out_hbm, acc_sh,
               src_v, dst_v, zb_v, rd_v, *bufs_sems):
        bufs = bufs_sems[:NBUF]
        gsem = bufs_sems[NBUF:2 * NBUF]
        ssem = bufs_sems[2 * NBUF:]
        c = lax.axis_index("c")
        s = lax.axis_index("s")
        wid = c * NS + s

        pltpu.sync_copy(zeros_hbm, zb_v)
        pltpu.sync_copy(srcb_hbm.at[wid], src_v)
        pltpu.sync_copy(dstb_hbm.at[wid], dst_v)

        @pl.loop(0, RPT // 16)
        def _zero(k):
            pltpu.sync_copy(zb_v, acc_sh.at[pl.ds(s * RPT + k * 16, 16)])

        plsc.subcore_barrier()

        def gather_start(j, b):
            pltpu.async_copy(g_hbm.at[src_v.at[j]], bufs[b], gsem[b])

        def gather_wait(b):
            pltpu.make_async_copy(g_hbm.at[src_v.at[0]], bufs[b], gsem[b]).wait()

        def scat_start(j, b):
            pltpu.async_copy(bufs[b], acc_sh.at[dst_v.at[j]], ssem[b], add=True)

        def scat_wait(b):
            pltpu.make_async_copy(
                bufs[b], acc_sh.at[dst_v.at[0]], ssem[b]
            ).wait()

        for b in range(NBUF):
            gather_start(b, b)

        @pl.loop(0, NBLK // NBUF - 1)
        def _edges(jg):
            j0 = jg * NBUF
            for b in range(NBUF):
                gather_wait(b)
                scat_start(j0 + b, b)
            for b in range(NBUF):
                scat_wait(b)
                gather_start(j0 + NBUF + b, b)

        for b in range(NBUF):
            gather_wait(b)
            scat_start(NBLK - NBUF + b, b)
        for b in range(NBUF):
            scat_wait(b)

        plsc.subcore_barrier()

        @pl.loop(0, 4)
        def _readout(k):
            r0 = s * RPT + k * (RPT // 4)
            pltpu.sync_copy(acc_sh.at[pl.ds(r0, RPT // 4)], rd_v)
            pltpu.sync_copy(rd_v, out_hbm.at[c].at[pl.ds(r0, RPT // 4)])

    return sc_hist, sc_agg


def _dinv_from_degp(degp_blk):
    deg = degp_blk[0, :, 0:1] + degp_blk[1, :, 0:1] + 1.0
    return lax.rsqrt(deg)


def _tc_prep1_body(x_ref, w1_ref, degp_ref, g1_ref):
    h = jnp.dot(x_ref[...], w1_ref[...], preferred_element_type=jnp.float32)
    g1_ref[...] = h * _dinv_from_degp(degp_ref[...])


def _tc_mid_body(p_ref, g1_ref, degp_ref, b1_ref, w2_ref, g2_ref):
    dinv = _dinv_from_degp(degp_ref[...])
    acc = p_ref[0] + p_ref[1] + g1_ref[...]
    out1 = jnp.maximum(acc * dinv + b1_ref[...], 0.0)
    h2 = jnp.dot(out1, w2_ref[...], preferred_element_type=jnp.float32)
    g2_ref[...] = h2 * dinv


def _tc_final_body(p_ref, g2_ref, degp_ref, b2_ref, batch_ref, wl_ref, bl_ref,
                   out_ref, sums_ref, cnt_ref):
    i = pl.program_id(0)
    dinv = _dinv_from_degp(degp_ref[...])
    acc = p_ref[0] + p_ref[1] + g2_ref[...]
    out2 = acc * dinv + b2_ref[...]

    rows = i * RB + lax.broadcasted_iota(jnp.int32, (RB, 1), 0)
    valid = rows < N
    out2m = jnp.where(valid, out2, 0.0)
    vones = jnp.where(valid, 1.0, 0.0)

    gids = lax.broadcasted_iota(jnp.int32, (RB, G), 1)
    oh = jnp.where((batch_ref[...] == gids) & valid, 1.0, 0.0)

    dn = (((0,), (0,)), ((), ()))
    part = lax.dot_general(oh, out2m, dn, preferred_element_type=jnp.float32)
    cntp = lax.dot_general(oh, vones, dn, preferred_element_type=jnp.float32)

    @pl.when(i == 0)
    def _init():
        sums_ref[...] = jnp.zeros_like(sums_ref)
        cnt_ref[...] = jnp.zeros_like(cnt_ref)

    sums_ref[...] += part
    cnt_ref[:, 0:1] += cntp

    @pl.when(i == (N_PAD // RB) - 1)
    def _fin():
        pooled = sums_ref[...] / jnp.maximum(cnt_ref[:, 0:1], 1.0)
        res = jnp.dot(pooled, wl_ref[...], preferred_element_type=jnp.float32)
        out_ref[...] = res + bl_ref[...]


def _row_spec(shape_tail):
    return pl.BlockSpec((RB,) + shape_tail, lambda i: (i,) + (0,) * len(shape_tail))


def _full_spec(shape):
    nd = len(shape)
    return pl.BlockSpec(shape, lambda i: (0,) * nd)


_degp_spec = pl.BlockSpec((2, RB, DH), lambda i: (0, i, 0))
_p_spec = pl.BlockSpec((2, RB, H), lambda i: (0, i, 0))
_grid = (N_PAD // RB,)


def kernel(x, edge_index, batch, W1, b1, W2, b2, Wl, bl):
    src = edge_index[0].astype(jnp.int32)
    dst = edge_index[1].astype(jnp.int32)
    batch2d = batch.astype(jnp.int32)[:, None]

    npad = EPAD - E
    pad_ids = jnp.arange(npad, dtype=jnp.int32)
    src_p = jnp.concatenate([src, pad_ids % N]).reshape(NW, NBLK, BLK)
    dst_p = jnp.concatenate([dst, N + pad_ids % (N_PAD - N)]).reshape(NW, NBLK, BLK)

    ones_upd = jnp.ones((BLK, DH), jnp.float32)
    zeros_h = jnp.zeros((16, DH), jnp.float32)
    zeros_a = jnp.zeros((16, H), jnp.float32)

    sc_hist, sc_agg = _sc_kernels()
    degp = sc_hist(dst_p, ones_upd, zeros_h)

    g1 = pl.pallas_call(
        _tc_prep1_body,
        grid=_grid,
        in_specs=[
            _row_spec((F_IN,)),
            _full_spec((F_IN, H)),
            _degp_spec,
        ],
        out_specs=_row_spec((H,)),
        out_shape=jax.ShapeDtypeStruct((N_PAD, H), jnp.float32),
    )(x, W1, degp)

    p1 = sc_agg(g1, src_p, dst_p, zeros_a)

    g2 = pl.pallas_call(
        _tc_mid_body,
        grid=_grid,
        in_specs=[
            _p_spec,
            _row_spec((H,)),
            _degp_spec,
            _full_spec((1, H)),
            _full_spec((H, H)),
        ],
        out_specs=_row_spec((H,)),
        out_shape=jax.ShapeDtypeStruct((N_PAD, H), jnp.float32),
    )(p1, g1, degp, b1[None, :], W2)

    p2 = sc_agg(g2, src_p, dst_p, zeros_a)

    out = pl.pallas_call(
        _tc_final_body,
        grid=_grid,
        in_specs=[
            _p_spec,
            _row_spec((H,)),
            _degp_spec,
            _full_spec((1, H)),
            _row_spec((1,)),
            _full_spec((H, 1)),
            _full_spec((1, 1)),
        ],
        out_specs=_full_spec((G, 1)),
        out_shape=jax.ShapeDtypeStruct((G, 1), jnp.float32),
        scratch_shapes=[
            pltpu.VMEM((G, G), jnp.float32),
            pltpu.VMEM((G, 8), jnp.float32),
        ],
    )(p2, g2, degp, b2[None, :], batch2d, Wl, bl[:, None])

    return out


# speedup vs baseline: 40.0916x; 1.0004x over previous
import functools

import jax
import jax.numpy as jnp
from jax import lax
from jax.experimental import pallas as pl
from jax.experimental.pallas import tpu as pltpu
from jax.experimental.pallas import tpu_sc as plsc

N = 10000
E = 320000
F_IN = 128
H = 64
G = 64

NC = 2
NS = 16
NW = NC * NS
BLK = 128
EPW = E // NW
NBUF = 4
NBLK = 80
EPAD = NW * NBLK * BLK
N_PAD = 10240
RPT = N_PAD // NS
DH = 16
RB = 1280

@functools.lru_cache(maxsize=1)
def _sc_kernels():
    mesh = plsc.VectorSubcoreMesh(
        core_axis_name="c", subcore_axis_name="s", num_cores=NC, num_subcores=NS
    )

    @functools.partial(
        pl.kernel,
        out_type=jax.ShapeDtypeStruct((NC, N_PAD, DH), jnp.float32),
        mesh=mesh,
        scratch_types=[
            pltpu.VMEM_SHARED((N_PAD, DH), jnp.float32),
            pltpu.VMEM((NBLK, BLK), jnp.int32),
            pltpu.VMEM((BLK, DH), jnp.float32),
            pltpu.VMEM((16, DH), jnp.float32),
            pltpu.VMEM((RPT, DH), jnp.float32),
            pltpu.SemaphoreType.DMA,
        ],
        compiler_params=pltpu.CompilerParams(use_tc_tiling_on_sc=False),
    )
    def sc_hist(dstb_hbm, ones_hbm, zeros_hbm, out_hbm, acc_sh,
                dst_v, ones_v, zb_v, rd_v, sem):
        c = lax.axis_index("c")
        s = lax.axis_index("s")
        wid = c * NS + s

        pltpu.sync_copy(ones_hbm, ones_v)
        pltpu.sync_copy(zeros_hbm, zb_v)
        pltpu.sync_copy(dstb_hbm.at[wid], dst_v)

        @pl.loop(0, RPT // 16)
        def _zero(k):
            pltpu.sync_copy(zb_v, acc_sh.at[pl.ds(s * RPT + k * 16, 16)])

        plsc.subcore_barrier()

        @pl.loop(0, NBLK // 8)
        def _scatter(j8):
            for b in range(8):
                pltpu.async_copy(ones_v, acc_sh.at[dst_v.at[j8 * 8 + b]], sem,
                                 add=True)
            for b in range(8):
                pltpu.make_async_copy(
                    ones_v, acc_sh.at[dst_v.at[0]], sem
                ).wait()

        plsc.subcore_barrier()

        pltpu.sync_copy(acc_sh.at[pl.ds(s * RPT, RPT)], rd_v)
        pltpu.sync_copy(rd_v, out_hbm.at[c].at[pl.ds(s * RPT, RPT)])

    @functools.partial(
        pl.kernel,
        out_type=jax.ShapeDtypeStruct((NC, N_PAD, H), jnp.float32),
        mesh=mesh,
        scratch_types=[
            pltpu.VMEM_SHARED((N_PAD, H), jnp.float32),
            pltpu.VMEM((NBLK, BLK), jnp.int32),
            pltpu.VMEM((NBLK, BLK), jnp.int32),
            pltpu.VMEM((16, H), jnp.float32),
            pltpu.VMEM((RPT // 4, H), jnp.float32),
        ]
        + [pltpu.VMEM((BLK, H), jnp.float32) for _ in range(NBUF)]
        + [pltpu.SemaphoreType.DMA for _ in range(2 * NBUF)],
        compiler_params=pltpu.CompilerParams(use_tc_tiling_on_sc=False),
    )
    def sc_agg(g_hbm, srcb_hbm, dstb_hbm, zeros_hbm, out_hbm, acc_sh,
               src_v, dst_v, zb_v, rd_v, *bufs_sems):
        bufs = bufs_sems[:NBUF]
        gsem = bufs_sems[NBUF:2 * NBUF]
        ssem = bufs_sems[2 * NBUF:]
        c = lax.axis_index("c")
        s = lax.axis_index("s")
        wid = c * NS + s

        pltpu.sync_copy(zeros_hbm, zb_v)
        pltpu.sync_copy(srcb_hbm.at[wid], src_v)
        pltpu.sync_copy(dstb_hbm.at[wid], dst_v)

        @pl.loop(0, RPT // 16)
        def _zero(k):
            pltpu.sync_copy(zb_v, acc_sh.at[pl.ds(s * RPT + k * 16, 16)])

        plsc.subcore_barrier()

        def gather_start(j, b):
            pltpu.async_copy(g_hbm.at[src_v.at[j]], bufs[b], gsem[b])

        def gather_wait(b):
            pltpu.make_async_copy(g_hbm.at[src_v.at[0]], bufs[b], gsem[b]).wait()

        def scat_start(j, b):
            pltpu.async_copy(bufs[b], acc_sh.at[dst_v.at[j]], ssem[b], add=True)

        def scat_wait(b):
            pltpu.make_async_copy(
                bufs[b], acc_sh.at[dst_v.at[0]], ssem[b]
            ).wait()

        for b in range(NBUF):
            gather_start(b, b)

        @pl.loop(0, NBLK // NBUF - 1)
        def _edges(jg):
            j0 = jg * NBUF
            for b in range(NBUF):
                gather_wait(b)
                scat_start(j0 + b, b)
            for b in range(NBUF):
                scat_wait(b)
                gather_start(j0 + NBUF + b, b)

        for b in range(NBUF):
            gather_wait(b)
            scat_start(NBLK - NBUF + b, b)
        for b in range(NBUF):
            scat_wait(b)

        plsc.subcore_barrier()

        @pl.loop(0, 4)
        def _readout(k):
            r0 = s * RPT + k * (RPT // 4)
            pltpu.sync_copy(acc_sh.at[pl.ds(r0, RPT // 4)], rd_v)
            pltpu.sync_copy(rd_v, out_hbm.at[c].at[pl.ds(r0, RPT // 4)])

    return sc_hist, sc_agg


def _dinv_from_degp(degp_blk):
    deg = degp_blk[0, :, 0:1] + degp_blk[1, :, 0:1] + 1.0
    return lax.rsqrt(deg)


def _tc_prep1_body(x_ref, w1_ref, degp_ref, g1_ref):
    h = jnp.dot(x_ref[...], w1_ref[...], preferred_element_type=jnp.float32)
    g1_ref[...] = h * _dinv_from_degp(degp_ref[...])


def _tc_mid_body(p_ref, g1_ref, degp_ref, b1_ref, w2_ref, g2_ref):
    dinv = _dinv_from_degp(degp_ref[...])
    acc = p_ref[0] + p_ref[1] + g1_ref[...]
    out1 = jnp.maximum(acc * dinv + b1_ref[...], 0.0)
    h2 = jnp.dot(out1, w2_ref[...], preferred_element_type=jnp.float32)
    g2_ref[...] = h2 * dinv


def _tc_final_body(p_ref, g2_ref, degp_ref, b2_ref, batch_ref, wl_ref, bl_ref,
                   out_ref, sums_ref, cnt_ref):
    i = pl.program_id(0)
    dinv = _dinv_from_degp(degp_ref[...])
    acc = p_ref[0] + p_ref[1] + g2_ref[...]
    out2 = acc * dinv + b2_ref[...]

    rows = i * RB + lax.broadcasted_iota(jnp.int32, (RB, 1), 0)
    valid = rows < N
    out2m = jnp.where(valid, out2, 0.0)
    vones = jnp.where(valid, 1.0, 0.0)

    gids = lax.broadcasted_iota(jnp.int32, (RB, G), 1)
    oh = jnp.where((batch_ref[...] == gids) & valid, 1.0, 0.0)

    dn = (((0,), (0,)), ((), ()))
    part = lax.dot_general(oh, out2m, dn, preferred_element_type=jnp.float32)
    cntp = lax.dot_general(oh, vones, dn, preferred_element_type=jnp.float32)

    @pl.when(i == 0)
    def _init():
        sums_ref[...] = jnp.zeros_like(sums_ref)
        cnt_ref[...] = jnp.zeros_like(cnt_ref)

    sums_ref[...] += part
    cnt_ref[:, 0:1] += cntp

    @pl.when(i == (N_PAD // RB) - 1)
    def _fin():
        pooled = sums_ref[...] / jnp.maximum(cnt_ref[:, 0:1], 1.0)
        res = jnp.dot(pooled, wl_ref[...], preferred_element_type=jnp.float32)
        out_ref[...] = res + bl_ref[...]


def _row_spec(shape_tail):
    return pl.BlockSpec((RB,) + shape_tail, lambda i: (i,) + (0,) * len(shape_tail))


def _full_spec(shape):
    nd = len(shape)
    return pl.BlockSpec(shape, lambda i: (0,) * nd)


_degp_spec = pl.BlockSpec((2, RB, DH), lambda i: (0, i, 0))
_p_spec = pl.BlockSpec((2, RB, H), lambda i: (0, i, 0))
_grid = (N_PAD // RB,)


def kernel(x, edge_index, batch, W1, b1, W2, b2, Wl, bl):
    src = edge_index[0].astype(jnp.int32)
    dst = edge_index[1].astype(jnp.int32)
    batch2d = batch.astype(jnp.int32)[:, None]

    npad = EPAD - E
    pad_ids = jnp.arange(npad, dtype=jnp.int32)
    src_p = jnp.concatenate([src, pad_ids % N]).reshape(NW, NBLK, BLK)
    dst_p = jnp.concatenate([dst, N + pad_ids % (N_PAD - N)]).reshape(NW, NBLK, BLK)

    ones_upd = jnp.ones((BLK, DH), jnp.float32)
    zeros_h = jnp.zeros((16, DH), jnp.float32)
    zeros_a = jnp.zeros((16, H), jnp.float32)

    sc_hist, sc_agg = _sc_kernels()
    degp = sc_hist(dst_p, ones_upd, zeros_h)

    g1 = pl.pallas_call(
        _tc_prep1_body,
        grid=_grid,
        in_specs=[
            _row_spec((F_IN,)),
            _full_spec((F_IN, H)),
            _degp_spec,
        ],
        out_specs=_row_spec((H,)),
        out_shape=jax.ShapeDtypeStruct((N_PAD, H), jnp.float32),
    )(x, W1, degp)

    p1 = sc_agg(g1, src_p, dst_p, zeros_a)

    g2 = pl.pallas_call(
        _tc_mid_body,
        grid=_grid,
        in_specs=[
            _p_spec,
            _row_spec((H,)),
            _degp_spec,
            _full_spec((1, H)),
            _full_spec((H, H)),
        ],
        out_specs=_row_spec((H,)),
        out_shape=jax.ShapeDtypeStruct((N_PAD, H), jnp.float32),
    )(p1, g1, degp, b1[None, :], W2)

    p2 = sc_agg(g2, src_p, dst_p, zeros_a)

    out = pl.pallas_call(
        _tc_final_body,
        grid=_grid,
        in_specs=[
            _p_spec,
            _row_spec((H,)),
            _degp_spec,
            _full_spec((1, H)),
            _row_spec((1,)),
            _full_spec((H, 1)),
            _full_spec((1, 1)),
        ],
        out_specs=_full_spec((G, 1)),
        out_shape=jax.ShapeDtypeStruct((G, 1), jnp.float32),
        scratch_shapes=[
            pltpu.VMEM((G, G), jnp.float32),
            pltpu.VMEM((G, 8), jnp.float32),
        ],
    )(p2, g2, degp, b2[None, :], batch2d, Wl, bl[:, None])

    return out


# speedup vs baseline: 46.6854x; 1.1645x over previous
import functools

import jax
import jax.numpy as jnp
from jax import lax
from jax.experimental import pallas as pl
from jax.experimental.pallas import tpu as pltpu
from jax.experimental.pallas import tpu_sc as plsc

N = 10000
E = 320000
F_IN = 128
H = 64
G = 64

NC = 2
NS = 16
NW = NC * NS
BLK = 128
EPW = E // NW
NBUF = 4
NBLK = 80
EPAD = NW * NBLK * BLK
N_PAD = 10240
RPT = N_PAD // NS
DH = 16
RB = 1280

@functools.lru_cache(maxsize=1)
def _sc_kernels():
    mesh = plsc.VectorSubcoreMesh(
        core_axis_name="c", subcore_axis_name="s", num_cores=NC, num_subcores=NS
    )

    @functools.partial(
        pl.kernel,
        out_type=jax.ShapeDtypeStruct((NC, N_PAD, 128), jnp.float32),
        mesh=mesh,
        scratch_types=[
            pltpu.VMEM_SHARED((N_PAD, DH), jnp.float32),
            pltpu.VMEM((NBLK, BLK), jnp.int32),
            pltpu.VMEM((BLK, DH), jnp.float32),
            pltpu.VMEM((16, DH), jnp.float32),
            pltpu.VMEM((RPT // 4, DH), jnp.float32),
            pltpu.SemaphoreType.DMA,
        ],
        compiler_params=pltpu.CompilerParams(use_tc_tiling_on_sc=False),
    )
    def sc_hist(dstb_hbm, ones_hbm, zeros_hbm, out_hbm, acc_sh,
                dst_v, ones_v, zb_v, rd_v, sem):
        c = lax.axis_index("c")
        s = lax.axis_index("s")
        wid = c * NS + s

        pltpu.sync_copy(ones_hbm, ones_v)
        pltpu.sync_copy(zeros_hbm, zb_v)
        pltpu.sync_copy(dstb_hbm.at[wid], dst_v)

        @pl.loop(0, RPT // 16)
        def _zero(k):
            pltpu.sync_copy(zb_v, acc_sh.at[pl.ds(s * RPT + k * 16, 16)])

        plsc.subcore_barrier()

        @pl.loop(0, NBLK // 8)
        def _scatter(j8):
            for b in range(8):
                pltpu.async_copy(ones_v, acc_sh.at[dst_v.at[j8 * 8 + b]], sem,
                                 add=True)
            for b in range(8):
                pltpu.make_async_copy(
                    ones_v, acc_sh.at[dst_v.at[0]], sem
                ).wait()

        plsc.subcore_barrier()

        @pl.loop(0, 4)
        def _readout(k):
            r0 = s * RPT + k * (RPT // 4)
            pltpu.sync_copy(acc_sh.at[pl.ds(r0, RPT // 4)], rd_v)
            pltpu.sync_copy(rd_v, out_hbm.at[c, pl.ds(r0, RPT // 4), pl.ds(0, DH)])

    @functools.partial(
        pl.kernel,
        out_type=jax.ShapeDtypeStruct((NC, N_PAD, 128), jnp.float32),
        mesh=mesh,
        scratch_types=[
            pltpu.VMEM_SHARED((N_PAD, H), jnp.float32),
            pltpu.VMEM((NBLK, BLK), jnp.int32),
            pltpu.VMEM((NBLK, BLK), jnp.int32),
            pltpu.VMEM((16, H), jnp.float32),
            pltpu.VMEM((RPT // 4, H), jnp.float32),
        ]
        + [pltpu.VMEM((BLK, H), jnp.float32) for _ in range(NBUF)]
        + [pltpu.SemaphoreType.DMA for _ in range(2 * NBUF)],
        compiler_params=pltpu.CompilerParams(use_tc_tiling_on_sc=False),
    )
    def sc_agg(g_hbm, srcb_hbm, dstb_hbm, zeros_hbm, out_hbm, acc_sh,
               src_v, dst_v, zb_v, rd_v, *bufs_sems):
        bufs = bufs_sems[:NBUF]
        gsem = bufs_sems[NBUF:2 * NBUF]
        ssem = bufs_sems[2 * NBUF:]
        c = lax.axis_index("c")
        s = lax.axis_index("s")
        wid = c * NS + s

        pltpu.sync_copy(zeros_hbm, zb_v)
        pltpu.sync_copy(srcb_hbm.at[wid], src_v)
        pltpu.sync_copy(dstb_hbm.at[wid], dst_v)

        @pl.loop(0, RPT // 16)
        def _zero(k):
            pltpu.sync_copy(zb_v, acc_sh.at[pl.ds(s * RPT + k * 16, 16)])

        plsc.subcore_barrier()

        def gather_start(j, b):
            pltpu.async_copy(g_hbm.at[src_v.at[j]], bufs[b], gsem[b])

        def gather_wait(b):
            pltpu.make_async_copy(g_hbm.at[src_v.at[0]], bufs[b], gsem[b]).wait()

        def scat_start(j, b):
            pltpu.async_copy(bufs[b], acc_sh.at[dst_v.at[j]], ssem[b], add=True)

        def scat_wait(b):
            pltpu.make_async_copy(
                bufs[b], acc_sh.at[dst_v.at[0]], ssem[b]
            ).wait()

        for b in range(NBUF):
            gather_start(b, b)

        @pl.loop(0, NBLK // NBUF - 1)
        def _edges(jg):
            j0 = jg * NBUF
            for b in range(NBUF):
                gather_wait(b)
                scat_start(j0 + b, b)
            for b in range(NBUF):
                scat_wait(b)
                gather_start(j0 + NBUF + b, b)

        for b in range(NBUF):
            gather_wait(b)
            scat_start(NBLK - NBUF + b, b)
        for b in range(NBUF):
            scat_wait(b)

        plsc.subcore_barrier()

        @pl.loop(0, 4)
        def _readout(k):
            r0 = s * RPT + k * (RPT // 4)
            pltpu.sync_copy(acc_sh.at[pl.ds(r0, RPT // 4)], rd_v)
            pltpu.sync_copy(rd_v, out_hbm.at[c, pl.ds(r0, RPT // 4), pl.ds(0, H)])

    return sc_hist, sc_agg


def _dinv_from_degp(da_ref, db_ref):
    deg = da_ref[0, :, 0:1] + db_ref[0, :, 0:1] + 1.0
    return lax.rsqrt(deg)


def _tc_prep1_body(x_ref, w1_ref, da_ref, db_ref, g1_ref):
    h = jnp.dot(x_ref[...], w1_ref[...], preferred_element_type=jnp.float32)
    g1_ref[:, 0:H] = h * _dinv_from_degp(da_ref, db_ref)


def _tc_mid_body(pa_ref, pb_ref, g1_ref, da_ref, db_ref, b1_ref, w2_ref,
                 g2_ref):
    dinv = _dinv_from_degp(da_ref, db_ref)
    acc = pa_ref[0, :, 0:H] + pb_ref[0, :, 0:H] + g1_ref[:, 0:H]
    out1 = jnp.maximum(acc * dinv + b1_ref[...], 0.0)
    h2 = jnp.dot(out1, w2_ref[...], preferred_element_type=jnp.float32)
    g2_ref[:, 0:H] = h2 * dinv


def _tc_final_body(pa_ref, pb_ref, g2_ref, da_ref, db_ref, b2_ref, batch_ref,
                   wl_ref, bl_ref, out_ref, sums_ref, cnt_ref):
    i = pl.program_id(0)
    dinv = _dinv_from_degp(da_ref, db_ref)
    acc = pa_ref[0, :, 0:H] + pb_ref[0, :, 0:H] + g2_ref[:, 0:H]
    out2 = acc * dinv + b2_ref[...]

    rows = i * RB + lax.broadcasted_iota(jnp.int32, (RB, 1), 0)
    valid = rows < N
    out2m = jnp.where(valid, out2, 0.0)
    vones = jnp.where(valid, 1.0, 0.0)

    gids = lax.broadcasted_iota(jnp.int32, (RB, G), 1)
    oh = jnp.where((batch_ref[...] == gids) & valid, 1.0, 0.0)

    dn = (((0,), (0,)), ((), ()))
    part = lax.dot_general(oh, out2m, dn, preferred_element_type=jnp.float32)
    cntp = lax.dot_general(oh, vones, dn, preferred_element_type=jnp.float32)

    @pl.when(i == 0)
    def _init():
        sums_ref[...] = jnp.zeros_like(sums_ref)
        cnt_ref[...] = jnp.zeros_like(cnt_ref)

    sums_ref[...] += part
    cnt_ref[:, 0:1] += cntp

    @pl.when(i == (N_PAD // RB) - 1)
    def _fin():
        pooled = sums_ref[...] / jnp.maximum(cnt_ref[:, 0:1], 1.0)
        res = jnp.dot(pooled, wl_ref[...], preferred_element_type=jnp.float32)
        out_ref[...] = res + bl_ref[...]


def _row_spec(shape_tail):
    return pl.BlockSpec((RB,) + shape_tail, lambda i: (i,) + (0,) * len(shape_tail))


def _full_spec(shape):
    nd = len(shape)
    return pl.BlockSpec(shape, lambda i: (0,) * nd)


_core0_spec = pl.BlockSpec((1, RB, 128), lambda i: (0, i, 0))
_core1_spec = pl.BlockSpec((1, RB, 128), lambda i: (1, i, 0))
_grid = (N_PAD // RB,)


def kernel(x, edge_index, batch, W1, b1, W2, b2, Wl, bl):
    src = edge_index[0].astype(jnp.int32)
    dst = edge_index[1].astype(jnp.int32)
    batch2d = batch.astype(jnp.int32)[:, None]

    npad = EPAD - E
    pad_ids = jnp.arange(npad, dtype=jnp.int32)
    src_p = (jnp.concatenate([src, pad_ids % N]) * 2).reshape(NW, NBLK, BLK)
    dst_p = jnp.concatenate([dst, N + pad_ids % (N_PAD - N)]).reshape(NW, NBLK, BLK)

    ones_upd = jnp.ones((BLK, DH), jnp.float32)
    zeros_h = jnp.zeros((16, DH), jnp.float32)
    zeros_a = jnp.zeros((16, H), jnp.float32)

    sc_hist, sc_agg = _sc_kernels()
    degp = sc_hist(dst_p, ones_upd, zeros_h)

    r128 = _row_spec((128,))

    g1 = pl.pallas_call(
        _tc_prep1_body,
        grid=_grid,
        in_specs=[
            _row_spec((F_IN,)),
            _full_spec((F_IN, H)),
            _core0_spec,
            _core1_spec,
        ],
        out_specs=r128,
        out_shape=jax.ShapeDtypeStruct((N_PAD, 128), jnp.float32),
    )(x, W1, degp, degp)

    p1 = sc_agg(g1.reshape(2 * N_PAD, H), src_p, dst_p, zeros_a)

    g2 = pl.pallas_call(
        _tc_mid_body,
        grid=_grid,
        in_specs=[
            _core0_spec,
            _core1_spec,
            r128,
            _core0_spec,
            _core1_spec,
            _full_spec((1, H)),
            _full_spec((H, H)),
        ],
        out_specs=r128,
        out_shape=jax.ShapeDtypeStruct((N_PAD, 128), jnp.float32),
    )(p1, p1, g1, degp, degp, b1[None, :], W2)

    p2 = sc_agg(g2.reshape(2 * N_PAD, H), src_p, dst_p, zeros_a)

    out = pl.pallas_call(
        _tc_final_body,
        grid=_grid,
        in_specs=[
            _core0_spec,
            _core1_spec,
            r128,
            _core0_spec,
            _core1_spec,
            _full_spec((1, H)),
            _row_spec((1,)),
            _full_spec((H, 1)),
            _full_spec((1, 1)),
        ],
        out_specs=_full_spec((G, 1)),
        out_shape=jax.ShapeDtypeStruct((G, 1), jnp.float32),
        scratch_shapes=[
            pltpu.VMEM((G, G), jnp.float32),
            pltpu.VMEM((G, 8), jnp.float32),
        ],
    )(p2, p2, g2, degp, degp, b2[None, :], batch2d, Wl, bl[:, None])

    return out
